# Initial kernel scaffold; baseline (speedup 1.0000x reference)
import jax
import jax.numpy as jnp
from jax.experimental import pallas as pl


def kernel(tgt, ious, bboxes, gather_mask, W1, b1, W2, b2, W3, b3, W4, b4, W5, b5):
    raise NotImplementedError("write your pallas kernel here")


# speedup vs baseline: 13.1928x; 13.1928x over previous
import functools

import numpy as np
import jax
import jax.numpy as jnp
from jax import lax
from jax.experimental import pallas as pl
from jax.experimental.pallas import tpu as pltpu
from jax.experimental.pallas import tpu_sc as plsc

_N = 5000
_K = 10
_DM = 256
_THR = 0.4
_EPS = 1e-7


_BR = 200


def _topk_body(iou_ref, maskrow_ref, negcol_ref, vals_ref, idx_ref):
    ov = (iou_ref[...] * maskrow_ref[...]) * negcol_ref[...]
    colid = lax.broadcasted_iota(jnp.int32, ov.shape, 1)
    work = ov
    big = jnp.int32(_N)
    for k in range(_K):
        m = jnp.max(work, axis=1, keepdims=True)
        j = jnp.min(jnp.where(work == m, colid, big), axis=1, keepdims=True)
        vals_ref[:, k : k + 1] = m
        idx_ref[:, k : k + 1] = j
        work = jnp.where(colid == j, jnp.float32(-1.0), work)


def _topk_call(ious2, mask_row, neg_col):
    grid = (_N // _BR,)
    return pl.pallas_call(
        _topk_body,
        grid=grid,
        in_specs=[
            pl.BlockSpec((_BR, _N), lambda i: (i, 0)),
            pl.BlockSpec((1, _N), lambda i: (0, 0)),
            pl.BlockSpec((_BR, 1), lambda i: (i, 0)),
        ],
        out_specs=(
            pl.BlockSpec((_BR, _K), lambda i: (i, 0)),
            pl.BlockSpec((_BR, _K), lambda i: (i, 0)),
        ),
        out_shape=(
            jax.ShapeDtypeStruct((_N, _K), jnp.float32),
            jax.ShapeDtypeStruct((_N, _K), jnp.int32),
        ),
    )(ious2, mask_row, neg_col)


_NC, _NS = 2, 16
_NW = _NC * _NS
_CH = 112
_NCHUNK = 14
_BPW = _CH * _NCHUNK
_B = _BPW * _NW
_TW = 128


def _sc_gather_body(table_hbm, idx_hbm, out_hbm, idx_v, rows_v, sem):
    wid = lax.axis_index("s") * _NC + lax.axis_index("c")
    pltpu.sync_copy(idx_hbm.at[wid], idx_v)
    for c in range(_NCHUNK):
        pltpu.async_copy(table_hbm.at[idx_v.at[c]], rows_v, sem).wait()
        pltpu.sync_copy(rows_v,
                        out_hbm.at[pl.ds(wid * _BPW + c * _CH, _CH)])


def _sc_gather_call(table, idx2d):
    mesh = plsc.VectorSubcoreMesh(core_axis_name="c", subcore_axis_name="s")
    fn = pl.kernel(
        _sc_gather_body,
        out_type=jax.ShapeDtypeStruct((_B, _TW), jnp.float32),
        mesh=mesh,
        scratch_types=[
            pltpu.VMEM((_NCHUNK, _CH), jnp.int32),
            pltpu.VMEM((_CH, _TW), jnp.float32),
            pltpu.SemaphoreType.DMA,
        ],
    )
    return fn(table, idx2d)


_BR2 = 1000


def _mlp_body(tgt_ref, rf_ref, vals_ref, g_ref, w1_ref, b1_ref, w2_ref, b2_ref,
              w3_ref, w3over_ref, b3_ref, w4_ref, b4_ref, w5_ref, b5_ref,
              invd_ref, out_ref):
    rf = rf_ref[...]
    cx, cy = rf[:, 0:1], rf[:, 1:2]
    w, h = rf[:, 2:3], rf[:, 3:4]
    neg = 1.0 - rf[:, 4:5]
    invd = invd_ref[...]
    scale = jnp.float32(2.0 * np.pi)
    eps = jnp.float32(_EPS)

    acc = None
    for k in range(_K):
        g = g_ref[:, _TW * k : _TW * (k + 1)]
        v = vals_ref[:, k : k + 1]
        nmk = g[:, 4:5]
        m1g = g[:, 5:6]
        ind = (v >= jnp.float32(_THR)).astype(jnp.float32)
        mk = nmk * ind
        p0 = jnp.log(jnp.maximum(jnp.abs(g[:, 0:1] - cx), eps))
        p1 = jnp.log(jnp.maximum(jnp.abs(g[:, 1:2] - cy), eps))
        p2 = jnp.log(jnp.maximum(jnp.abs(g[:, 2:3] - w), eps))
        p3 = jnp.log(jnp.maximum(jnp.abs(g[:, 3:4] - h), eps))
        s = (neg * m1g) * mk
        f0 = jnp.sin((scale * p0) * invd) * s
        f1 = jnp.cos((scale * p1) * invd) * s
        f2 = jnp.sin((scale * p2) * invd) * s
        f3 = jnp.cos((scale * p3) * invd) * s
        pre = (v * mk) * w3over_ref[...]
        pre = pre + jnp.dot(f0, w3_ref[64:192, :], preferred_element_type=jnp.float32)
        pre = pre + jnp.dot(f1, w3_ref[192:320, :], preferred_element_type=jnp.float32)
        pre = pre + jnp.dot(f2, w3_ref[320:448, :], preferred_element_type=jnp.float32)
        pre = pre + jnp.dot(f3, w3_ref[448:576, :], preferred_element_type=jnp.float32)
        h1 = jax.nn.relu(pre + b3_ref[...])
        f = jnp.dot(h1, w4_ref[...], preferred_element_type=jnp.float32) + b4_ref[...]
        contrib = f * mk
        acc = contrib if acc is None else jnp.maximum(acc, contrib)

    tgt = tgt_ref[...]
    c1 = jax.nn.relu(jnp.dot(tgt, w1_ref[...], preferred_element_type=jnp.float32) + b1_ref[...])
    cur = jnp.dot(c1, w2_ref[...], preferred_element_type=jnp.float32) + b2_ref[...]
    ct = cur * neg + acc
    o = jax.nn.relu(jnp.dot(ct, w5_ref[...], preferred_element_type=jnp.float32) + b5_ref[...])
    out_ref[...] = o * neg


def _mlp_call(tgt2, table, vals, gflat, W1, b1, W2, b2, W3, w3over, b3, W4, b4,
              W5, b5, invd):
    grid = (_N // _BR2,)
    fixed = lambda i: (0, 0)
    return pl.pallas_call(
        _mlp_body,
        grid=grid,
        in_specs=[
            pl.BlockSpec((_BR2, _DM), lambda i: (i, 0)),
            pl.BlockSpec((_BR2, _TW), lambda i: (i, 0)),
            pl.BlockSpec((_BR2, _K), lambda i: (i, 0)),
            pl.BlockSpec((_BR2, _K * _TW), lambda i: (i, 0)),
            pl.BlockSpec((_DM, _DM), fixed),
            pl.BlockSpec((1, _DM), fixed),
            pl.BlockSpec((_DM, _DM), fixed),
            pl.BlockSpec((1, _DM), fixed),
            pl.BlockSpec((2 * _DM + 64, _DM), fixed),
            pl.BlockSpec((1, _DM), fixed),
            pl.BlockSpec((1, _DM), fixed),
            pl.BlockSpec((_DM, _DM), fixed),
            pl.BlockSpec((1, _DM), fixed),
            pl.BlockSpec((_DM, _DM), fixed),
            pl.BlockSpec((1, _DM), fixed),
            pl.BlockSpec((1, 128), fixed),
        ],
        out_specs=pl.BlockSpec((_BR2, _DM), lambda i: (i, 0)),
        out_shape=jax.ShapeDtypeStruct((_N, _DM), jnp.float32),
    )(tgt2, table, vals, gflat, W1, b1, W2, b2, W3, w3over, b3, W4, b4, W5, b5,
      invd)


def kernel(tgt, ious, bboxes, gather_mask, W1, b1, W2, b2, W3, b3, W4, b4, W5, b5):
    mask = gather_mask[0, :, 0]
    negm = 1.0 - mask
    bx = bboxes[0]
    ctr = 0.5 * (bx[:, 2:4] + bx[:, 0:2])
    hw = bx[:, 2:4] - bx[:, 0:2]
    one_minus_neg = 1.0 - negm
    table = jnp.concatenate(
        [ctr, hw, mask[:, None], one_minus_neg[:, None],
         jnp.zeros((_N, _TW - 6), jnp.float32)], axis=1)

    vals, idx = _topk_call(ious[0], mask[None, :], negm[:, None])

    idxf = jnp.concatenate(
        [idx.reshape(-1), jnp.zeros((_B - _N * _K,), jnp.int32)])
    gath = _sc_gather_call(table, idxf.reshape(_NW, _NCHUNK, _CH))
    gflat = gath[: _N * _K].reshape(_N, _K * _TW)

    dim_t = jnp.arange(128, dtype=jnp.float32)
    dim_t = jnp.float32(10000.0) ** (2.0 * jnp.floor(dim_t / 2.0) / 128.0)
    invd = (1.0 / dim_t).reshape(1, 128)
    w3over = jnp.sum(W3[:64, :], axis=0, keepdims=True)

    out = _mlp_call(tgt[0], table, vals, gflat, W1, b1.reshape(1, -1), W2,
                    b2.reshape(1, -1), W3, w3over, b3.reshape(1, -1), W4,
                    b4.reshape(1, -1), W5, b5.reshape(1, -1), invd)
    return out[None]


# speedup vs baseline: 14.3088x; 1.0846x over previous
import functools

import numpy as np
import jax
import jax.numpy as jnp
from jax import lax
from jax.experimental import pallas as pl
from jax.experimental.pallas import tpu as pltpu
from jax.experimental.pallas import tpu_sc as plsc

_N = 5000
_K = 10
_DM = 256
_THR = 0.4
_EPS = 1e-7


_BR = 200


def _topk_body(iou_ref, maskrow_ref, negcol_ref, vals_ref, idx_ref):
    ov = (iou_ref[...] * maskrow_ref[...]) * negcol_ref[...]
    colid = lax.broadcasted_iota(jnp.int32, ov.shape, 1)
    work = ov
    big = jnp.int32(_N)
    for k in range(_K):
        m = jnp.max(work, axis=1, keepdims=True)
        j = jnp.min(jnp.where(work == m, colid, big), axis=1, keepdims=True)
        vals_ref[:, k : k + 1] = m
        idx_ref[:, k : k + 1] = j
        work = jnp.where(colid == j, jnp.float32(-1.0), work)


def _topk_call(ious2, mask_row, neg_col):
    grid = (_N // _BR,)
    return pl.pallas_call(
        _topk_body,
        grid=grid,
        in_specs=[
            pl.BlockSpec((_BR, _N), lambda i: (i, 0)),
            pl.BlockSpec((1, _N), lambda i: (0, 0)),
            pl.BlockSpec((_BR, 1), lambda i: (i, 0)),
        ],
        out_specs=(
            pl.BlockSpec((_BR, _K), lambda i: (i, 0)),
            pl.BlockSpec((_BR, _K), lambda i: (i, 0)),
        ),
        out_shape=(
            jax.ShapeDtypeStruct((_N, _K), jnp.float32),
            jax.ShapeDtypeStruct((_N, _K), jnp.int32),
        ),
    )(ious2, mask_row, neg_col)


_NC, _NS = 2, 16
_NW = _NC * _NS
_CH = 112
_NCHUNK = 14
_BPW = _CH * _NCHUNK
_B = _BPW * _NW
_TW = 128


def _sc_gather_body(table_hbm, idx_hbm, out_hbm, idx_v, rows_v, sem):
    wid = lax.axis_index("s") * _NC + lax.axis_index("c")
    pltpu.sync_copy(idx_hbm.at[wid], idx_v)
    for c in range(_NCHUNK):
        pltpu.async_copy(table_hbm.at[idx_v.at[c]], rows_v, sem).wait()
        pltpu.sync_copy(rows_v,
                        out_hbm.at[pl.ds(wid * _BPW + c * _CH, _CH)])


def _sc_gather_call(table, idx2d):
    mesh = plsc.VectorSubcoreMesh(core_axis_name="c", subcore_axis_name="s")
    fn = pl.kernel(
        _sc_gather_body,
        out_type=jax.ShapeDtypeStruct((_B, _TW), jnp.float32),
        mesh=mesh,
        scratch_types=[
            pltpu.VMEM((_NCHUNK, _CH), jnp.int32),
            pltpu.VMEM((_CH, _TW), jnp.float32),
            pltpu.SemaphoreType.DMA,
        ],
    )
    return fn(table, idx2d)


_BRC = 1000


def _cur_body(tgt_ref, rf_ref, w1_ref, b1_ref, w2_ref, b2_ref, out_ref):
    neg = 1.0 - rf_ref[:, 4:5]
    c1 = jax.nn.relu(jnp.dot(tgt_ref[...], w1_ref[...],
                             preferred_element_type=jnp.float32) + b1_ref[...])
    cur = jnp.dot(c1, w2_ref[...], preferred_element_type=jnp.float32) + b2_ref[...]
    out_ref[...] = cur * neg


def _cur_call(tgt2, table, W1, b1, W2, b2):
    fixed = lambda i: (0, 0)
    return pl.pallas_call(
        _cur_body,
        grid=(_N // _BRC,),
        in_specs=[
            pl.BlockSpec((_BRC, _DM), lambda i: (i, 0)),
            pl.BlockSpec((_BRC, _TW), lambda i: (i, 0)),
            pl.BlockSpec((_DM, _DM), fixed),
            pl.BlockSpec((1, _DM), fixed),
            pl.BlockSpec((_DM, _DM), fixed),
            pl.BlockSpec((1, _DM), fixed),
        ],
        out_specs=pl.BlockSpec((_BRC, _DM), lambda i: (i, 0)),
        out_shape=jax.ShapeDtypeStruct((_N, _DM), jnp.float32),
    )(tgt2, table, W1, b1, W2, b2)


_BR2 = 200


def _mlp_body(rf_ref, vals_ref, g_ref, w3_ref, w3over_ref, b3_ref, w4_ref,
              b4_ref, cur_ref, w5_ref, b5_ref, invd_ref, out_ref):
    rf = rf_ref[...]
    cx, cy = rf[:, 0:1], rf[:, 1:2]
    w, h = rf[:, 2:3], rf[:, 3:4]
    neg = 1.0 - rf[:, 4:5]
    invd = invd_ref[...]
    scale = jnp.float32(2.0 * np.pi)
    eps = jnp.float32(_EPS)

    feats, vmks, mks = [], [], []
    for k in range(_K):
        g = g_ref[:, _TW * k : _TW * (k + 1)]
        v = vals_ref[:, k : k + 1]
        nmk = g[:, 4:5]
        m1g = g[:, 5:6]
        ind = (v >= jnp.float32(_THR)).astype(jnp.float32)
        mk = nmk * ind
        p0 = jnp.log(jnp.maximum(jnp.abs(g[:, 0:1] - cx), eps))
        p1 = jnp.log(jnp.maximum(jnp.abs(g[:, 1:2] - cy), eps))
        p2 = jnp.log(jnp.maximum(jnp.abs(g[:, 2:3] - w), eps))
        p3 = jnp.log(jnp.maximum(jnp.abs(g[:, 3:4] - h), eps))
        s = (neg * m1g) * mk
        f0 = jnp.sin((scale * p0) * invd) * s
        f1 = jnp.cos((scale * p1) * invd) * s
        f2 = jnp.sin((scale * p2) * invd) * s
        f3 = jnp.cos((scale * p3) * invd) * s
        feats.append(jnp.concatenate([f0, f1, f2, f3], axis=1))
        vmks.append(v * mk)
        mks.append(mk)

    F = jnp.concatenate(feats, axis=0)
    VMK = jnp.concatenate(vmks, axis=0)
    pre = jnp.dot(F, w3_ref[64:, :], preferred_element_type=jnp.float32)
    pre = pre + VMK * w3over_ref[...] + b3_ref[...]
    h1 = jax.nn.relu(pre)
    f = jnp.dot(h1, w4_ref[...], preferred_element_type=jnp.float32) + b4_ref[...]

    acc = f[0:_BR2, :] * mks[0]
    for k in range(1, _K):
        acc = jnp.maximum(acc, f[k * _BR2 : (k + 1) * _BR2, :] * mks[k])

    ct = cur_ref[...] + acc
    o = jax.nn.relu(jnp.dot(ct, w5_ref[...], preferred_element_type=jnp.float32) + b5_ref[...])
    out_ref[...] = o * neg


def _mlp_call(table, vals, gflat, W3, w3over, b3, W4, b4, cur, W5, b5, invd):
    grid = (_N // _BR2,)
    fixed = lambda i: (0, 0)
    return pl.pallas_call(
        _mlp_body,
        grid=grid,
        in_specs=[
            pl.BlockSpec((_BR2, _TW), lambda i: (i, 0)),
            pl.BlockSpec((_BR2, _K), lambda i: (i, 0)),
            pl.BlockSpec((_BR2, _K * _TW), lambda i: (i, 0)),
            pl.BlockSpec((2 * _DM + 64, _DM), fixed),
            pl.BlockSpec((1, _DM), fixed),
            pl.BlockSpec((1, _DM), fixed),
            pl.BlockSpec((_DM, _DM), fixed),
            pl.BlockSpec((1, _DM), fixed),
            pl.BlockSpec((_BR2, _DM), lambda i: (i, 0)),
            pl.BlockSpec((_DM, _DM), fixed),
            pl.BlockSpec((1, _DM), fixed),
            pl.BlockSpec((1, 128), fixed),
        ],
        out_specs=pl.BlockSpec((_BR2, _DM), lambda i: (i, 0)),
        out_shape=jax.ShapeDtypeStruct((_N, _DM), jnp.float32),
    )(table, vals, gflat, W3, w3over, b3, W4, b4, cur, W5, b5, invd)


def kernel(tgt, ious, bboxes, gather_mask, W1, b1, W2, b2, W3, b3, W4, b4, W5, b5):
    mask = gather_mask[0, :, 0]
    negm = 1.0 - mask
    bx = bboxes[0]
    ctr = 0.5 * (bx[:, 2:4] + bx[:, 0:2])
    hw = bx[:, 2:4] - bx[:, 0:2]
    one_minus_neg = 1.0 - negm
    table = jnp.concatenate(
        [ctr, hw, mask[:, None], one_minus_neg[:, None],
         jnp.zeros((_N, _TW - 6), jnp.float32)], axis=1)

    vals, idx = _topk_call(ious[0], mask[None, :], negm[:, None])

    idxf = jnp.concatenate(
        [idx.reshape(-1), jnp.zeros((_B - _N * _K,), jnp.int32)])
    gath = _sc_gather_call(table, idxf.reshape(_NW, _NCHUNK, _CH))
    gflat = gath[: _N * _K].reshape(_N, _K * _TW)

    dim_t = jnp.arange(128, dtype=jnp.float32)
    dim_t = jnp.float32(10000.0) ** (2.0 * jnp.floor(dim_t / 2.0) / 128.0)
    invd = (1.0 / dim_t).reshape(1, 128)
    w3over = jnp.sum(W3[:64, :], axis=0, keepdims=True)

    cur = _cur_call(tgt[0], table, W1, b1.reshape(1, -1), W2, b2.reshape(1, -1))
    out = _mlp_call(table, vals, gflat, W3, w3over, b3.reshape(1, -1), W4,
                    b4.reshape(1, -1), cur, W5, b5.reshape(1, -1), invd)
    return out[None]


# speedup vs baseline: 23.0960x; 1.6141x over previous
import functools

import numpy as np
import jax
import jax.numpy as jnp
from jax import lax
from jax.experimental import pallas as pl
from jax.experimental.pallas import tpu as pltpu
from jax.experimental.pallas import tpu_sc as plsc

_N = 5000
_K = 10
_DM = 256
_THR = 0.4
_EPS = 1e-7


_BR = 200


def _topk_body(iou_ref, maskrow_ref, negcol_ref, vals_ref, idx_ref):
    ov = (iou_ref[...] * maskrow_ref[...]) * negcol_ref[...]
    colf = lax.broadcasted_iota(jnp.int32, ov.shape, 1).astype(jnp.float32)
    work = ov
    big = jnp.float32(_N)
    for k in range(_K):
        m = jnp.max(work, axis=1, keepdims=True)
        j = jnp.min(jnp.where(work == m, colf, big), axis=1, keepdims=True)
        vals_ref[:, k : k + 1] = m
        idx_ref[:, k : k + 1] = j.astype(jnp.int32)
        work = jnp.where(colf == j, jnp.float32(-1.0), work)


def _topk_call(ious2, mask_row, neg_col):
    grid = (_N // _BR,)
    return pl.pallas_call(
        _topk_body,
        grid=grid,
        in_specs=[
            pl.BlockSpec((_BR, _N), lambda i: (i, 0)),
            pl.BlockSpec((1, _N), lambda i: (0, 0)),
            pl.BlockSpec((_BR, 1), lambda i: (i, 0)),
        ],
        out_specs=(
            pl.BlockSpec((_BR, _K), lambda i: (i, 0)),
            pl.BlockSpec((_BR, _K), lambda i: (i, 0)),
        ),
        out_shape=(
            jax.ShapeDtypeStruct((_N, _K), jnp.float32),
            jax.ShapeDtypeStruct((_N, _K), jnp.int32),
        ),
    )(ious2, mask_row, neg_col)


_NC, _NS = 2, 16
_NW = _NC * _NS
_CH = 112
_NCHUNK = 14
_BPW = _CH * _NCHUNK
_B = _BPW * _NW
_TW = 128


def _sc_gather_body(table_hbm, idx_hbm, out_hbm, idx_v, rows_v, sem):
    wid = lax.axis_index("s") * _NC + lax.axis_index("c")
    pltpu.sync_copy(idx_hbm.at[wid], idx_v)
    for c in range(_NCHUNK):
        pltpu.async_copy(table_hbm.at[idx_v.at[c]], rows_v, sem).wait()
        pltpu.sync_copy(rows_v,
                        out_hbm.at[pl.ds(wid * _BPW + c * _CH, _CH)])


def _sc_gather_call(table, idx2d):
    mesh = plsc.VectorSubcoreMesh(core_axis_name="c", subcore_axis_name="s")
    fn = pl.kernel(
        _sc_gather_body,
        out_type=jax.ShapeDtypeStruct((_B, _TW), jnp.float32),
        mesh=mesh,
        scratch_types=[
            pltpu.VMEM((_NCHUNK, _CH), jnp.int32),
            pltpu.VMEM((_CH, _TW), jnp.float32),
            pltpu.SemaphoreType.DMA,
        ],
    )
    return fn(table, idx2d)


_BRC = 1000


def _cur_body(tgt_ref, rf_ref, w1_ref, b1_ref, w2_ref, b2_ref, out_ref):
    neg = 1.0 - rf_ref[:, 4:5]
    c1 = jax.nn.relu(jnp.dot(tgt_ref[...], w1_ref[...],
                             preferred_element_type=jnp.float32) + b1_ref[...])
    cur = jnp.dot(c1, w2_ref[...], preferred_element_type=jnp.float32) + b2_ref[...]
    out_ref[...] = cur * neg


def _cur_call(tgt2, table, W1, b1, W2, b2):
    fixed = lambda i: (0, 0)
    return pl.pallas_call(
        _cur_body,
        grid=(_N // _BRC,),
        in_specs=[
            pl.BlockSpec((_BRC, _DM), lambda i: (i, 0)),
            pl.BlockSpec((_BRC, _TW), lambda i: (i, 0)),
            pl.BlockSpec((_DM, _DM), fixed),
            pl.BlockSpec((1, _DM), fixed),
            pl.BlockSpec((_DM, _DM), fixed),
            pl.BlockSpec((1, _DM), fixed),
        ],
        out_specs=pl.BlockSpec((_BRC, _DM), lambda i: (i, 0)),
        out_shape=jax.ShapeDtypeStruct((_N, _DM), jnp.float32),
    )(tgt2, table, W1, b1, W2, b2)


_BR2 = 200

_INV2PI = 0.15915494309189535
_CW1 = np.float32(6.2831855)
_CW2 = np.float32(2.0 * np.pi - 6.283185482025146)
_SIN_C = (-0.16666553, 0.008332403, -1.9808633e-4, 2.6997138e-6, -2.0362212e-8)
_COS_C = (-0.49999989, 0.041666489, -1.3887804e-3, 2.4769883e-5, -2.7079024e-7,
          1.7245067e-9)


def _reduce_2pi(x):
    n = jnp.round(x * jnp.float32(_INV2PI))
    return (x - n * _CW1) - n * _CW2


def _sin_poly(x):
    r = _reduce_2pi(x)
    t = r * r
    p = jnp.float32(_SIN_C[4])
    for c in (_SIN_C[3], _SIN_C[2], _SIN_C[1], _SIN_C[0], 0.9999996):
        p = p * t + jnp.float32(c)
    return p * r


def _cos_poly(x):
    r = _reduce_2pi(x)
    t = r * r
    p = jnp.float32(_COS_C[5])
    for c in (_COS_C[4], _COS_C[3], _COS_C[2], _COS_C[1], _COS_C[0], 0.99999999):
        p = p * t + jnp.float32(c)
    return p


def _mlp_body(rf_ref, vals_ref, g_ref, w3_ref, w3over_ref, b3_ref, w4_ref,
              b4_ref, cur_ref, w5_ref, b5_ref, invd_ref, out_ref):
    rf = rf_ref[...]
    cx, cy = rf[:, 0:1], rf[:, 1:2]
    w, h = rf[:, 2:3], rf[:, 3:4]
    neg = 1.0 - rf[:, 4:5]
    invd = invd_ref[...]
    scale = jnp.float32(2.0 * np.pi)
    eps = jnp.float32(_EPS)

    feats, vmks, mks = [], [], []
    for k in range(_K):
        g = g_ref[:, _TW * k : _TW * (k + 1)]
        v = vals_ref[:, k : k + 1]
        nmk = g[:, 4:5]
        m1g = g[:, 5:6]
        ind = (v >= jnp.float32(_THR)).astype(jnp.float32)
        mk = nmk * ind
        p0 = jnp.log(jnp.maximum(jnp.abs(g[:, 0:1] - cx), eps))
        p1 = jnp.log(jnp.maximum(jnp.abs(g[:, 1:2] - cy), eps))
        p2 = jnp.log(jnp.maximum(jnp.abs(g[:, 2:3] - w), eps))
        p3 = jnp.log(jnp.maximum(jnp.abs(g[:, 3:4] - h), eps))
        s = (neg * m1g) * mk
        f0 = _sin_poly((scale * p0) * invd) * s
        f1 = _cos_poly((scale * p1) * invd) * s
        f2 = _sin_poly((scale * p2) * invd) * s
        f3 = _cos_poly((scale * p3) * invd) * s
        feats.append(jnp.concatenate([f0, f1, f2, f3], axis=1))
        vmks.append(v * mk)
        mks.append(mk)

    F = jnp.concatenate(feats, axis=0)
    VMK = jnp.concatenate(vmks, axis=0)
    pre = jnp.dot(F, w3_ref[64:, :], preferred_element_type=jnp.float32)
    pre = pre + VMK * w3over_ref[...] + b3_ref[...]
    h1 = jax.nn.relu(pre)
    f = jnp.dot(h1, w4_ref[...], preferred_element_type=jnp.float32) + b4_ref[...]

    acc = f[0:_BR2, :] * mks[0]
    for k in range(1, _K):
        acc = jnp.maximum(acc, f[k * _BR2 : (k + 1) * _BR2, :] * mks[k])

    ct = cur_ref[...] + acc
    o = jax.nn.relu(jnp.dot(ct, w5_ref[...], preferred_element_type=jnp.float32) + b5_ref[...])
    out_ref[...] = o * neg


def _mlp_call(table, vals, gflat, W3, w3over, b3, W4, b4, cur, W5, b5, invd):
    grid = (_N // _BR2,)
    fixed = lambda i: (0, 0)
    return pl.pallas_call(
        _mlp_body,
        grid=grid,
        in_specs=[
            pl.BlockSpec((_BR2, _TW), lambda i: (i, 0)),
            pl.BlockSpec((_BR2, _K), lambda i: (i, 0)),
            pl.BlockSpec((_BR2, _K * _TW), lambda i: (i, 0)),
            pl.BlockSpec((2 * _DM + 64, _DM), fixed),
            pl.BlockSpec((1, _DM), fixed),
            pl.BlockSpec((1, _DM), fixed),
            pl.BlockSpec((_DM, _DM), fixed),
            pl.BlockSpec((1, _DM), fixed),
            pl.BlockSpec((_BR2, _DM), lambda i: (i, 0)),
            pl.BlockSpec((_DM, _DM), fixed),
            pl.BlockSpec((1, _DM), fixed),
            pl.BlockSpec((1, 128), fixed),
        ],
        out_specs=pl.BlockSpec((_BR2, _DM), lambda i: (i, 0)),
        out_shape=jax.ShapeDtypeStruct((_N, _DM), jnp.float32),
    )(table, vals, gflat, W3, w3over, b3, W4, b4, cur, W5, b5, invd)


def kernel(tgt, ious, bboxes, gather_mask, W1, b1, W2, b2, W3, b3, W4, b4, W5, b5):
    mask = gather_mask[0, :, 0]
    negm = 1.0 - mask
    bx = bboxes[0]
    ctr = 0.5 * (bx[:, 2:4] + bx[:, 0:2])
    hw = bx[:, 2:4] - bx[:, 0:2]
    one_minus_neg = 1.0 - negm
    table = jnp.concatenate(
        [ctr, hw, mask[:, None], one_minus_neg[:, None],
         jnp.zeros((_N, _TW - 6), jnp.float32)], axis=1)

    vals, idx = _topk_call(ious[0], mask[None, :], negm[:, None])

    idxf = jnp.concatenate(
        [idx.reshape(-1), jnp.zeros((_B - _N * _K,), jnp.int32)])
    gath = _sc_gather_call(table, idxf.reshape(_NW, _NCHUNK, _CH))
    gflat = gath[: _N * _K].reshape(_N, _K * _TW)

    dim_t = jnp.arange(128, dtype=jnp.float32)
    dim_t = jnp.float32(10000.0) ** (2.0 * jnp.floor(dim_t / 2.0) / 128.0)
    invd = (1.0 / dim_t).reshape(1, 128)
    w3over = jnp.sum(W3[:64, :], axis=0, keepdims=True)

    cur = _cur_call(tgt[0], table, W1, b1.reshape(1, -1), W2, b2.reshape(1, -1))
    out = _mlp_call(table, vals, gflat, W3, w3over, b3.reshape(1, -1), W4,
                    b4.reshape(1, -1), cur, W5, b5.reshape(1, -1), invd)
    return out[None]
